# baseline (speedup 1.0000x reference)
import functools

import jax
import jax.numpy as jnp
from jax.experimental import pallas as pl
from jax.experimental.pallas import tpu as pltpu
from jax.experimental.pallas import tpu_sc as plsc

E_BLK = 2000
T_BLK = 2000
_NC = 2

_SC_MESH = plsc.VectorSubcoreMesh(core_axis_name="c", subcore_axis_name="s")


def _sc_gather(table, idx):
    t_n = idx.shape[0]
    h = table.shape[1]
    w = 128
    idx2 = idx.reshape(1, t_n)

    @functools.partial(
        pl.kernel,
        out_type=jax.ShapeDtypeStruct((t_n, h), jnp.float32),
        mesh=_SC_MESH,
    )
    def k(x_hbm, i_hbm, o_hbm):
        def body(i_vmem, o_vmem):
            pltpu.sync_copy(x_hbm.at[i_vmem.at[0]], o_vmem)

        pltpu.emit_pipeline(
            body,
            grid=(t_n // w,),
            in_specs=[pl.BlockSpec((1, w), index_map=lambda i: (0, i))],
            out_specs=[pl.BlockSpec((w, h), index_map=lambda i: (i, 0))],
            core_axis_name=("c", "s"),
            dimension_semantics=(pltpu.PARALLEL,),
        )(i_hbm, o_hbm)

    return k(table, idx2)


def _silu(v):
    return v / (1.0 + jnp.exp(-v))


def _mm(a, w):
    return jax.lax.dot_general(
        a.astype(jnp.bfloat16), w, (((1,), (0,)), ((), ())),
        preferred_element_type=jnp.float32)


_GW = 16
_IW = 128
_NIW = 5
_MW = _IW * _NIW


def _sc_scatter_add(ts, idxs, e):
    nchunk = len(ts)
    h = ts[0].shape[1]
    n_groups_per_core = h // (2 * _GW)
    nwins = [t.shape[0] // _MW for t in ts]
    stripe = e // 16
    zrows = stripe // ((stripe + _MW - 1) // _MW)
    assert stripe % zrows == 0 and zrows <= _MW
    nks = []
    for nwin in nwins:
        nk = (nwin + 15) // 16
        nks.append(nk + (nk % 2))
    idx2s = [idx.reshape(nw, _NIW, _IW) for idx, nw in zip(idxs, nwins)]

    @functools.partial(
        pl.kernel,
        out_type=jax.ShapeDtypeStruct((e, h), jnp.float32),
        mesh=_SC_MESH,
        compiler_params=pltpu.CompilerParams(use_tc_tiling_on_sc=False),
        scratch_types=[
            pltpu.VMEM_SHARED((e, _GW), jnp.float32),
            pltpu.VMEM((_NIW, _IW), jnp.int32),
            pltpu.VMEM((_NIW, _IW), jnp.int32),
            pltpu.VMEM((_MW, _GW), jnp.float32),
            pltpu.VMEM((_MW, _GW), jnp.float32),
            pltpu.SemaphoreType.DMA,
            pltpu.SemaphoreType.DMA,
            pltpu.SemaphoreType.DMA,
            pltpu.SemaphoreType.DMA,
        ],
    )
    def k(*refs):
        t_hbms = refs[:nchunk]
        i_hbms = refs[nchunk:2 * nchunk]
        (agg_hbm, acc, idx_v0, idx_v1, upd_v0, upd_v1,
         sem_i0, sem_i1, sem_u0, sem_u1) = refs[2 * nchunk:]
        c = jax.lax.axis_index("c")
        s = jax.lax.axis_index("s")
        idx_v = (idx_v0, idx_v1)
        upd_v = (upd_v0, upd_v1)
        sem_i = (sem_i0, sem_i1)
        sem_u = (sem_u0, sem_u1)
        base = s * stripe

        for g in range(n_groups_per_core):
            col = (c * n_groups_per_core + g) * _GW

            @pl.loop(0, zrows)
            def _(r):
                upd_v0[r, :] = jnp.zeros((_GW,), jnp.float32)

            for z in range(stripe // zrows):
                pltpu.sync_copy(upd_v0.at[pl.ds(0, zrows), :],
                                acc.at[pl.ds(base + z * zrows, zrows), :])
            plsc.subcore_barrier()

            for t_hbm, i_hbm, nwin, nk in zip(t_hbms, i_hbms, nwins, nks):
                def fetch(b, w, t_hbm=t_hbm, i_hbm=i_hbm):
                    pltpu.async_copy(i_hbm.at[w], idx_v[b], sem_i[b])
                    pltpu.async_copy(
                        t_hbm.at[pl.ds(w * _MW, _MW), pl.ds(col, _GW)],
                        upd_v[b], sem_u[b])

                fetch(0, s)

                @pl.loop(0, nk, step=2)
                def _(kk, t_hbm=t_hbm, i_hbm=i_hbm, nwin=nwin, fetch=fetch):
                    for b in range(2):
                        w = (kk + b) * 16 + s
                        nxt = (kk + b + 1) * 16 + s

                        @pl.when(w < nwin)
                        def _():
                            pltpu.make_async_copy(
                                i_hbm.at[w], idx_v[b], sem_i[b]).wait()
                            pltpu.make_async_copy(
                                t_hbm.at[pl.ds(w * _MW, _MW), pl.ds(col, _GW)],
                                upd_v[b], sem_u[b]).wait()

                            @pl.when(nxt < nwin)
                            def _():
                                fetch(1 - b, nxt)

                            for j in range(_NIW):
                                pltpu.sync_copy(
                                    upd_v[b].at[pl.ds(j * _IW, _IW), :],
                                    acc.at[idx_v[b].at[j]], add=True)

            plsc.subcore_barrier()
            pltpu.sync_copy(acc.at[pl.ds(base, stripe), :],
                            agg_hbm.at[pl.ds(base, stripe), pl.ds(col, _GW)])
            plsc.subcore_barrier()

    return k(*ts, *idx2s)


def _pre_body(x_ref, rbf_ref, wkj_ref, bkj_ref, wrbf_ref, xkj_ref):
    h = _silu(_mm(x_ref[...], wkj_ref[...]) + bkj_ref[...])
    rp = _mm(rbf_ref[...], wrbf_ref[...])
    xkj_ref[...] = h * rp


def _preamble(x, rbf, wkj, bkj, wrbf):
    e, h = x.shape
    nr = rbf.shape[1]
    nb = e // E_BLK
    return pl.pallas_call(
        _pre_body,
        grid=(nb,),
        in_specs=[
            pl.BlockSpec((E_BLK, h), lambda i: (i, 0)),
            pl.BlockSpec((E_BLK, nr), lambda i: (i, 0)),
            pl.BlockSpec((h, h), lambda i: (0, 0)),
            pl.BlockSpec((1, h), lambda i: (0, 0)),
            pl.BlockSpec((nr, h), lambda i: (0, 0)),
        ],
        out_specs=pl.BlockSpec((E_BLK, h), lambda i: (i, 0)),
        out_shape=jax.ShapeDtypeStruct((e, h), jnp.float32),
    )(x, rbf, wkj, bkj, wrbf)


def _bil_body(g_ref, sbf_ref, wsbf_ref, wt_ref, t_ref):
    sp = _mm(sbf_ref[...], wsbf_ref[...])
    g = g_ref[...].astype(jnp.bfloat16)
    nb = wt_ref.shape[0]
    acc = None
    for j in range(nb):
        m = jax.lax.dot_general(g, wt_ref[j], (((1,), (0,)), ((), ())),
                                preferred_element_type=jnp.float32)
        term = sp[:, j:j + 1] * m
        acc = term if acc is None else acc + term
    t_ref[...] = acc


def _bilinear(g, sbf, wsbf, wt):
    t_n, h = g.shape
    nsr = sbf.shape[1]
    nb = wt.shape[0]
    blocks = t_n // T_BLK
    return pl.pallas_call(
        _bil_body,
        grid=(blocks,),
        in_specs=[
            pl.BlockSpec((T_BLK, h), lambda i: (i, 0)),
            pl.BlockSpec((T_BLK, nsr), lambda i: (i, 0)),
            pl.BlockSpec((nsr, nb), lambda i: (0, 0)),
            pl.BlockSpec((nb, h, h), lambda i: (0, 0, 0)),
        ],
        out_specs=pl.BlockSpec((T_BLK, h), lambda i: (i, 0)),
        out_shape=jax.ShapeDtypeStruct((t_n, h), jnp.float32),
    )(g, sbf, wsbf, wt)


def _fin_body(n_agg, *refs):
    agg_refs = refs[:n_agg]
    (x_ref, wji_ref, bji_ref, w1_ref, b1_ref, w2_ref, b2_ref,
     wl_ref, bl_ref, wa_ref, ba_ref, wb_ref, bb_ref, wc_ref, bc_ref,
     wd_ref, bd_ref, h_ref) = refs[n_agg:]
    x = x_ref[...]
    x_ji = _silu(_mm(x, wji_ref[...]) + bji_ref[...])
    agg = agg_refs[0][...]
    for r in agg_refs[1:]:
        agg = agg + r[...]
    h = x_ji + agg
    h = h + _silu(_mm(_silu(_mm(h, w1_ref[...]) + b1_ref[...]), w2_ref[...])
                  + b2_ref[...])
    h = _silu(_mm(h, wl_ref[...]) + bl_ref[...]) + x
    h = h + _silu(_mm(_silu(_mm(h, wa_ref[...]) + ba_ref[...]), wb_ref[...])
                  + bb_ref[...])
    h = h + _silu(_mm(_silu(_mm(h, wc_ref[...]) + bc_ref[...]), wd_ref[...])
                  + bd_ref[...])
    h_ref[...] = h


def _final(aggs, x, mats, biases):
    e, h = x.shape
    nb = e // E_BLK
    mat_spec = pl.BlockSpec((h, h), lambda i: (0, 0))
    bias_spec = pl.BlockSpec((1, h), lambda i: (0, 0))
    edge_spec = pl.BlockSpec((E_BLK, h), lambda i: (i, 0))
    in_specs = [edge_spec] * (len(aggs) + 1)
    args = list(aggs) + [x]
    for m, b in zip(mats, biases):
        in_specs += [mat_spec, bias_spec]
        args += [m, b]
    return pl.pallas_call(
        functools.partial(_fin_body, len(aggs)),
        grid=(nb,),
        in_specs=in_specs,
        out_specs=pl.BlockSpec((E_BLK, h), lambda i: (i, 0)),
        out_shape=jax.ShapeDtypeStruct((e, h), jnp.float32),
    )(*args)


def kernel(x, rbf, sbf, idx_kj, idx_ji, W_rbf, W_sbf, W_kj, b_kj, W_ji, b_ji,
           W_bil, W_bs1, b_bs1, W_bs2, b_bs2, W_lin, b_lin,
           W_as1a, b_as1a, W_as1b, b_as1b, W_as2a, b_as2a, W_as2b, b_as2b):
    e = x.shape[0]
    t_n = idx_kj.shape[0]
    bf = jnp.bfloat16

    x_kj = _preamble(x, rbf, W_kj.astype(bf), b_kj.reshape(1, -1),
                     W_rbf.astype(bf))

    wt = jnp.transpose(W_bil, (1, 2, 0)).astype(bf)
    wsbf = W_sbf.astype(bf)

    unit = 16000
    n_units = t_n // unit
    base = n_units // _NC
    bounds = [0]
    for c in range(_NC):
        u = base + (1 if c < n_units % _NC else 0)
        bounds.append(bounds[-1] + u * unit)

    t_chunks = []
    idx_chunks = []
    for c in range(_NC):
        lo, hi = bounds[c], bounds[c + 1]
        g_c = _sc_gather(x_kj, idx_kj[lo:hi])
        t_chunks.append(_bilinear(g_c, sbf[lo:hi], wsbf, wt))
        idx_chunks.append(idx_ji[lo:hi])

    agg = _sc_scatter_add(t_chunks, idx_chunks, e)

    mats = (W_ji, W_bs1, W_bs2, W_lin, W_as1a, W_as1b, W_as2a, W_as2b)
    biases = (b_ji, b_bs1, b_bs2, b_lin, b_as1a, b_as1b, b_as2a, b_as2b)
    return _final([agg], x, tuple(m.astype(bf) for m in mats),
                  tuple(b.reshape(1, -1) for b in biases))


# speedup vs baseline: 1.0331x; 1.0331x over previous
import functools

import jax
import jax.numpy as jnp
from jax.experimental import pallas as pl
from jax.experimental.pallas import tpu as pltpu
from jax.experimental.pallas import tpu_sc as plsc

E_BLK = 2000
T_BLK = 2000
_NC = 3

_SC_MESH = plsc.VectorSubcoreMesh(core_axis_name="c", subcore_axis_name="s")


def _sc_gather(table, idx):
    t_n = idx.shape[0]
    h = table.shape[1]
    w = 128
    idx2 = idx.reshape(1, t_n)

    @functools.partial(
        pl.kernel,
        out_type=jax.ShapeDtypeStruct((t_n, h), jnp.float32),
        mesh=_SC_MESH,
    )
    def k(x_hbm, i_hbm, o_hbm):
        def body(i_vmem, o_vmem):
            pltpu.sync_copy(x_hbm.at[i_vmem.at[0]], o_vmem)

        pltpu.emit_pipeline(
            body,
            grid=(t_n // w,),
            in_specs=[pl.BlockSpec((1, w), index_map=lambda i: (0, i))],
            out_specs=[pl.BlockSpec((w, h), index_map=lambda i: (i, 0))],
            core_axis_name=("c", "s"),
            dimension_semantics=(pltpu.PARALLEL,),
        )(i_hbm, o_hbm)

    return k(table, idx2)


def _silu(v):
    return v / (1.0 + jnp.exp(-v))


def _mm(a, w):
    return jax.lax.dot_general(
        a.astype(jnp.bfloat16), w, (((1,), (0,)), ((), ())),
        preferred_element_type=jnp.float32)


_GW = 16
_IW = 128
_NIW = 5
_MW = _IW * _NIW


def _sc_scatter_add(ts, idxs, e):
    nchunk = len(ts)
    h = ts[0].shape[1]
    n_groups_per_core = h // (2 * _GW)
    nwins = [t.shape[0] // _MW for t in ts]
    stripe = e // 16
    zrows = stripe // ((stripe + _MW - 1) // _MW)
    assert stripe % zrows == 0 and zrows <= _MW
    nks = []
    for nwin in nwins:
        nk = (nwin + 15) // 16
        nks.append(nk + (nk % 2))
    idx2s = [idx.reshape(nw, _NIW, _IW) for idx, nw in zip(idxs, nwins)]

    @functools.partial(
        pl.kernel,
        out_type=jax.ShapeDtypeStruct((e, h), jnp.float32),
        mesh=_SC_MESH,
        compiler_params=pltpu.CompilerParams(use_tc_tiling_on_sc=False),
        scratch_types=[
            pltpu.VMEM_SHARED((e, _GW), jnp.float32),
            pltpu.VMEM((_NIW, _IW), jnp.int32),
            pltpu.VMEM((_NIW, _IW), jnp.int32),
            pltpu.VMEM((_MW, _GW), jnp.float32),
            pltpu.VMEM((_MW, _GW), jnp.float32),
            pltpu.SemaphoreType.DMA,
            pltpu.SemaphoreType.DMA,
            pltpu.SemaphoreType.DMA,
            pltpu.SemaphoreType.DMA,
        ],
    )
    def k(*refs):
        t_hbms = refs[:nchunk]
        i_hbms = refs[nchunk:2 * nchunk]
        (agg_hbm, acc, idx_v0, idx_v1, upd_v0, upd_v1,
         sem_i0, sem_i1, sem_u0, sem_u1) = refs[2 * nchunk:]
        c = jax.lax.axis_index("c")
        s = jax.lax.axis_index("s")
        idx_v = (idx_v0, idx_v1)
        upd_v = (upd_v0, upd_v1)
        sem_i = (sem_i0, sem_i1)
        sem_u = (sem_u0, sem_u1)
        base = s * stripe

        for g in range(n_groups_per_core):
            col = (c * n_groups_per_core + g) * _GW

            @pl.loop(0, zrows)
            def _(r):
                upd_v0[r, :] = jnp.zeros((_GW,), jnp.float32)

            for z in range(stripe // zrows):
                pltpu.sync_copy(upd_v0.at[pl.ds(0, zrows), :],
                                acc.at[pl.ds(base + z * zrows, zrows), :])
            plsc.subcore_barrier()

            for t_hbm, i_hbm, nwin, nk in zip(t_hbms, i_hbms, nwins, nks):
                def fetch(b, w, t_hbm=t_hbm, i_hbm=i_hbm):
                    pltpu.async_copy(i_hbm.at[w], idx_v[b], sem_i[b])
                    pltpu.async_copy(
                        t_hbm.at[pl.ds(w * _MW, _MW), pl.ds(col, _GW)],
                        upd_v[b], sem_u[b])

                fetch(0, s)

                @pl.loop(0, nk, step=2)
                def _(kk, t_hbm=t_hbm, i_hbm=i_hbm, nwin=nwin, fetch=fetch):
                    for b in range(2):
                        w = (kk + b) * 16 + s
                        nxt = (kk + b + 1) * 16 + s

                        @pl.when(w < nwin)
                        def _():
                            pltpu.make_async_copy(
                                i_hbm.at[w], idx_v[b], sem_i[b]).wait()
                            pltpu.make_async_copy(
                                t_hbm.at[pl.ds(w * _MW, _MW), pl.ds(col, _GW)],
                                upd_v[b], sem_u[b]).wait()

                            @pl.when(nxt < nwin)
                            def _():
                                fetch(1 - b, nxt)

                            for j in range(_NIW):
                                pltpu.sync_copy(
                                    upd_v[b].at[pl.ds(j * _IW, _IW), :],
                                    acc.at[idx_v[b].at[j]], add=True)

            plsc.subcore_barrier()
            pltpu.sync_copy(acc.at[pl.ds(base, stripe), :],
                            agg_hbm.at[pl.ds(base, stripe), pl.ds(col, _GW)])
            plsc.subcore_barrier()

    return k(*ts, *idx2s)


def _pre_body(x_ref, rbf_ref, wkj_ref, bkj_ref, wrbf_ref, xkj_ref):
    h = _silu(_mm(x_ref[...], wkj_ref[...]) + bkj_ref[...])
    rp = _mm(rbf_ref[...], wrbf_ref[...])
    xkj_ref[...] = h * rp


def _preamble(x, rbf, wkj, bkj, wrbf):
    e, h = x.shape
    nr = rbf.shape[1]
    nb = e // E_BLK
    return pl.pallas_call(
        _pre_body,
        grid=(nb,),
        in_specs=[
            pl.BlockSpec((E_BLK, h), lambda i: (i, 0)),
            pl.BlockSpec((E_BLK, nr), lambda i: (i, 0)),
            pl.BlockSpec((h, h), lambda i: (0, 0)),
            pl.BlockSpec((1, h), lambda i: (0, 0)),
            pl.BlockSpec((nr, h), lambda i: (0, 0)),
        ],
        out_specs=pl.BlockSpec((E_BLK, h), lambda i: (i, 0)),
        out_shape=jax.ShapeDtypeStruct((e, h), jnp.float32),
    )(x, rbf, wkj, bkj, wrbf)


def _bil_body(g_ref, sbf_ref, wsbf_ref, wt_ref, t_ref):
    sp = _mm(sbf_ref[...], wsbf_ref[...])
    g = g_ref[...].astype(jnp.bfloat16)
    nb = wt_ref.shape[0]
    acc = None
    for j in range(nb):
        m = jax.lax.dot_general(g, wt_ref[j], (((1,), (0,)), ((), ())),
                                preferred_element_type=jnp.float32)
        term = sp[:, j:j + 1] * m
        acc = term if acc is None else acc + term
    t_ref[...] = acc


def _bilinear(g, sbf, wsbf, wt):
    t_n, h = g.shape
    nsr = sbf.shape[1]
    nb = wt.shape[0]
    blocks = t_n // T_BLK
    return pl.pallas_call(
        _bil_body,
        grid=(blocks,),
        in_specs=[
            pl.BlockSpec((T_BLK, h), lambda i: (i, 0)),
            pl.BlockSpec((T_BLK, nsr), lambda i: (i, 0)),
            pl.BlockSpec((nsr, nb), lambda i: (0, 0)),
            pl.BlockSpec((nb, h, h), lambda i: (0, 0, 0)),
        ],
        out_specs=pl.BlockSpec((T_BLK, h), lambda i: (i, 0)),
        out_shape=jax.ShapeDtypeStruct((t_n, h), jnp.float32),
    )(g, sbf, wsbf, wt)


def _fin_body(n_agg, *refs):
    agg_refs = refs[:n_agg]
    (x_ref, wji_ref, bji_ref, w1_ref, b1_ref, w2_ref, b2_ref,
     wl_ref, bl_ref, wa_ref, ba_ref, wb_ref, bb_ref, wc_ref, bc_ref,
     wd_ref, bd_ref, h_ref) = refs[n_agg:]
    x = x_ref[...]
    x_ji = _silu(_mm(x, wji_ref[...]) + bji_ref[...])
    agg = agg_refs[0][...]
    for r in agg_refs[1:]:
        agg = agg + r[...]
    h = x_ji + agg
    h = h + _silu(_mm(_silu(_mm(h, w1_ref[...]) + b1_ref[...]), w2_ref[...])
                  + b2_ref[...])
    h = _silu(_mm(h, wl_ref[...]) + bl_ref[...]) + x
    h = h + _silu(_mm(_silu(_mm(h, wa_ref[...]) + ba_ref[...]), wb_ref[...])
                  + bb_ref[...])
    h = h + _silu(_mm(_silu(_mm(h, wc_ref[...]) + bc_ref[...]), wd_ref[...])
                  + bd_ref[...])
    h_ref[...] = h


def _final(aggs, x, mats, biases):
    e, h = x.shape
    nb = e // E_BLK
    mat_spec = pl.BlockSpec((h, h), lambda i: (0, 0))
    bias_spec = pl.BlockSpec((1, h), lambda i: (0, 0))
    edge_spec = pl.BlockSpec((E_BLK, h), lambda i: (i, 0))
    in_specs = [edge_spec] * (len(aggs) + 1)
    args = list(aggs) + [x]
    for m, b in zip(mats, biases):
        in_specs += [mat_spec, bias_spec]
        args += [m, b]
    return pl.pallas_call(
        functools.partial(_fin_body, len(aggs)),
        grid=(nb,),
        in_specs=in_specs,
        out_specs=pl.BlockSpec((E_BLK, h), lambda i: (i, 0)),
        out_shape=jax.ShapeDtypeStruct((e, h), jnp.float32),
    )(*args)


def kernel(x, rbf, sbf, idx_kj, idx_ji, W_rbf, W_sbf, W_kj, b_kj, W_ji, b_ji,
           W_bil, W_bs1, b_bs1, W_bs2, b_bs2, W_lin, b_lin,
           W_as1a, b_as1a, W_as1b, b_as1b, W_as2a, b_as2a, W_as2b, b_as2b):
    e = x.shape[0]
    t_n = idx_kj.shape[0]
    bf = jnp.bfloat16

    x_kj = _preamble(x, rbf, W_kj.astype(bf), b_kj.reshape(1, -1),
                     W_rbf.astype(bf))

    wt = jnp.transpose(W_bil, (1, 2, 0)).astype(bf)
    wsbf = W_sbf.astype(bf)

    unit = 16000
    n_units = t_n // unit
    base = n_units // _NC
    bounds = [0]
    for c in range(_NC):
        u = base + (1 if c < n_units % _NC else 0)
        bounds.append(bounds[-1] + u * unit)

    aggs = []
    for c in range(_NC):
        lo, hi = bounds[c], bounds[c + 1]
        g_c = _sc_gather(x_kj, idx_kj[lo:hi])
        t_c = _bilinear(g_c, sbf[lo:hi], wsbf, wt)
        aggs.append(_sc_scatter_add([t_c], [idx_ji[lo:hi]], e))

    mats = (W_ji, W_bs1, W_bs2, W_lin, W_as1a, W_as1b, W_as2a, W_as2b)
    biases = (b_ji, b_bs1, b_bs2, b_lin, b_as1a, b_as1b, b_as2a, b_as2b)
    return _final(aggs, x, tuple(m.astype(bf) for m in mats),
                  tuple(b.reshape(1, -1) for b in biases))


# speedup vs baseline: 1.0947x; 1.0597x over previous
import functools

import jax
import jax.numpy as jnp
from jax.experimental import pallas as pl
from jax.experimental.pallas import tpu as pltpu
from jax.experimental.pallas import tpu_sc as plsc

E_BLK = 2000
T_BLK = 2000
_NC = 2

_SC_MESH = plsc.VectorSubcoreMesh(core_axis_name="c", subcore_axis_name="s")


def _sc_gather(table, idx):
    t_n = idx.shape[0]
    h = table.shape[1]
    w = 128
    idx2 = idx.reshape(1, t_n)

    @functools.partial(
        pl.kernel,
        out_type=jax.ShapeDtypeStruct((t_n, h), jnp.float32),
        mesh=_SC_MESH,
    )
    def k(x_hbm, i_hbm, o_hbm):
        def body(i_vmem, o_vmem):
            pltpu.sync_copy(x_hbm.at[i_vmem.at[0]], o_vmem)

        pltpu.emit_pipeline(
            body,
            grid=(t_n // w,),
            in_specs=[pl.BlockSpec((1, w), index_map=lambda i: (0, i))],
            out_specs=[pl.BlockSpec((w, h), index_map=lambda i: (i, 0))],
            core_axis_name=("c", "s"),
            dimension_semantics=(pltpu.PARALLEL,),
        )(i_hbm, o_hbm)

    return k(table, idx2)


def _silu(v):
    return v / (1.0 + jnp.exp(-v))


def _mm(a, w):
    return jax.lax.dot_general(
        a.astype(jnp.bfloat16), w, (((1,), (0,)), ((), ())),
        preferred_element_type=jnp.float32)


_GW = 16
_IW = 128
_NIW = 5
_MW = _IW * _NIW


def _sc_scatter_add(ts, idxs, e):
    nchunk = len(ts)
    h = ts[0].shape[1]
    n_groups_per_core = h // (2 * _GW)
    nwins = [t.shape[0] // _MW for t in ts]
    stripe = e // 16
    zrows = stripe // ((stripe + _MW - 1) // _MW)
    assert stripe % zrows == 0 and zrows <= _MW
    nks = []
    for nwin in nwins:
        nk = (nwin + 15) // 16
        nks.append(nk + (nk % 2))
    idx2s = [idx.reshape(nw, _NIW, _IW) for idx, nw in zip(idxs, nwins)]

    @functools.partial(
        pl.kernel,
        out_type=jax.ShapeDtypeStruct((e, h), jnp.float32),
        mesh=_SC_MESH,
        compiler_params=pltpu.CompilerParams(use_tc_tiling_on_sc=False),
        scratch_types=[
            pltpu.VMEM_SHARED((e, _GW), jnp.float32),
            pltpu.VMEM((_NIW, _IW), jnp.int32),
            pltpu.VMEM((_NIW, _IW), jnp.int32),
            pltpu.VMEM((_MW, _GW), jnp.float32),
            pltpu.VMEM((_MW, _GW), jnp.float32),
            pltpu.SemaphoreType.DMA,
            pltpu.SemaphoreType.DMA,
            pltpu.SemaphoreType.DMA,
            pltpu.SemaphoreType.DMA,
        ],
    )
    def k(*refs):
        t_hbms = refs[:nchunk]
        i_hbms = refs[nchunk:2 * nchunk]
        (agg_hbm, acc, idx_v0, idx_v1, upd_v0, upd_v1,
         sem_i0, sem_i1, sem_u0, sem_u1) = refs[2 * nchunk:]
        c = jax.lax.axis_index("c")
        s = jax.lax.axis_index("s")
        idx_v = (idx_v0, idx_v1)
        upd_v = (upd_v0, upd_v1)
        sem_i = (sem_i0, sem_i1)
        sem_u = (sem_u0, sem_u1)
        base = s * stripe

        for g in range(n_groups_per_core):
            col = (c * n_groups_per_core + g) * _GW

            @pl.loop(0, zrows)
            def _(r):
                upd_v0[r, :] = jnp.zeros((_GW,), jnp.float32)

            for z in range(stripe // zrows):
                pltpu.async_copy(upd_v0.at[pl.ds(0, zrows), :],
                                 acc.at[pl.ds(base + z * zrows, zrows), :],
                                 sem_u0)
            for z in range(stripe // zrows):
                pltpu.make_async_copy(
                    upd_v0.at[pl.ds(0, zrows), :],
                    acc.at[pl.ds(base + z * zrows, zrows), :], sem_u0).wait()
            plsc.subcore_barrier()

            for t_hbm, i_hbm, nwin, nk in zip(t_hbms, i_hbms, nwins, nks):
                def fetch(b, w, t_hbm=t_hbm, i_hbm=i_hbm):
                    pltpu.async_copy(i_hbm.at[w], idx_v[b], sem_i[b])
                    pltpu.async_copy(
                        t_hbm.at[pl.ds(w * _MW, _MW), pl.ds(col, _GW)],
                        upd_v[b], sem_u[b])

                fetch(0, s)

                @pl.loop(0, nk, step=2)
                def _(kk, t_hbm=t_hbm, i_hbm=i_hbm, nwin=nwin, fetch=fetch):
                    for b in range(2):
                        w = (kk + b) * 16 + s
                        nxt = (kk + b + 1) * 16 + s

                        @pl.when(w < nwin)
                        def _():
                            pltpu.make_async_copy(
                                i_hbm.at[w], idx_v[b], sem_i[b]).wait()
                            pltpu.make_async_copy(
                                t_hbm.at[pl.ds(w * _MW, _MW), pl.ds(col, _GW)],
                                upd_v[b], sem_u[b]).wait()

                            @pl.when(nxt < nwin)
                            def _():
                                fetch(1 - b, nxt)

                            for j in range(_NIW):
                                pltpu.sync_copy(
                                    upd_v[b].at[pl.ds(j * _IW, _IW), :],
                                    acc.at[idx_v[b].at[j]], add=True)

            plsc.subcore_barrier()
            pltpu.sync_copy(acc.at[pl.ds(base, stripe), :],
                            agg_hbm.at[pl.ds(base, stripe), pl.ds(col, _GW)])
            if g == n_groups_per_core - 1:
                plsc.subcore_barrier()

    return k(*ts, *idx2s)


def _pre_body(x_ref, rbf_ref, wkj_ref, bkj_ref, wrbf_ref, xkj_ref):
    h = _silu(_mm(x_ref[...], wkj_ref[...]) + bkj_ref[...])
    rp = _mm(rbf_ref[...], wrbf_ref[...])
    xkj_ref[...] = h * rp


def _preamble(x, rbf, wkj, bkj, wrbf):
    e, h = x.shape
    nr = rbf.shape[1]
    nb = e // E_BLK
    return pl.pallas_call(
        _pre_body,
        grid=(nb,),
        in_specs=[
            pl.BlockSpec((E_BLK, h), lambda i: (i, 0)),
            pl.BlockSpec((E_BLK, nr), lambda i: (i, 0)),
            pl.BlockSpec((h, h), lambda i: (0, 0)),
            pl.BlockSpec((1, h), lambda i: (0, 0)),
            pl.BlockSpec((nr, h), lambda i: (0, 0)),
        ],
        out_specs=pl.BlockSpec((E_BLK, h), lambda i: (i, 0)),
        out_shape=jax.ShapeDtypeStruct((e, h), jnp.float32),
    )(x, rbf, wkj, bkj, wrbf)


def _bil_body(g_ref, sbf_ref, wsbf_ref, wt_ref, t_ref):
    sp = _mm(sbf_ref[...], wsbf_ref[...])
    g = g_ref[...].astype(jnp.bfloat16)
    nb = wt_ref.shape[0]
    acc = None
    for j in range(nb):
        m = jax.lax.dot_general(g, wt_ref[j], (((1,), (0,)), ((), ())),
                                preferred_element_type=jnp.float32)
        term = sp[:, j:j + 1] * m
        acc = term if acc is None else acc + term
    t_ref[...] = acc


def _bilinear(g, sbf, wsbf, wt):
    t_n, h = g.shape
    nsr = sbf.shape[1]
    nb = wt.shape[0]
    blocks = t_n // T_BLK
    return pl.pallas_call(
        _bil_body,
        grid=(blocks,),
        in_specs=[
            pl.BlockSpec((T_BLK, h), lambda i: (i, 0)),
            pl.BlockSpec((T_BLK, nsr), lambda i: (i, 0)),
            pl.BlockSpec((nsr, nb), lambda i: (0, 0)),
            pl.BlockSpec((nb, h, h), lambda i: (0, 0, 0)),
        ],
        out_specs=pl.BlockSpec((T_BLK, h), lambda i: (i, 0)),
        out_shape=jax.ShapeDtypeStruct((t_n, h), jnp.float32),
    )(g, sbf, wsbf, wt)


def _fin_body(n_agg, *refs):
    agg_refs = refs[:n_agg]
    (x_ref, wji_ref, bji_ref, w1_ref, b1_ref, w2_ref, b2_ref,
     wl_ref, bl_ref, wa_ref, ba_ref, wb_ref, bb_ref, wc_ref, bc_ref,
     wd_ref, bd_ref, h_ref) = refs[n_agg:]
    x = x_ref[...]
    x_ji = _silu(_mm(x, wji_ref[...]) + bji_ref[...])
    agg = agg_refs[0][...]
    for r in agg_refs[1:]:
        agg = agg + r[...]
    h = x_ji + agg
    h = h + _silu(_mm(_silu(_mm(h, w1_ref[...]) + b1_ref[...]), w2_ref[...])
                  + b2_ref[...])
    h = _silu(_mm(h, wl_ref[...]) + bl_ref[...]) + x
    h = h + _silu(_mm(_silu(_mm(h, wa_ref[...]) + ba_ref[...]), wb_ref[...])
                  + bb_ref[...])
    h = h + _silu(_mm(_silu(_mm(h, wc_ref[...]) + bc_ref[...]), wd_ref[...])
                  + bd_ref[...])
    h_ref[...] = h


def _final(aggs, x, mats, biases):
    e, h = x.shape
    nb = e // E_BLK
    mat_spec = pl.BlockSpec((h, h), lambda i: (0, 0))
    bias_spec = pl.BlockSpec((1, h), lambda i: (0, 0))
    edge_spec = pl.BlockSpec((E_BLK, h), lambda i: (i, 0))
    in_specs = [edge_spec] * (len(aggs) + 1)
    args = list(aggs) + [x]
    for m, b in zip(mats, biases):
        in_specs += [mat_spec, bias_spec]
        args += [m, b]
    return pl.pallas_call(
        functools.partial(_fin_body, len(aggs)),
        grid=(nb,),
        in_specs=in_specs,
        out_specs=pl.BlockSpec((E_BLK, h), lambda i: (i, 0)),
        out_shape=jax.ShapeDtypeStruct((e, h), jnp.float32),
    )(*args)


def kernel(x, rbf, sbf, idx_kj, idx_ji, W_rbf, W_sbf, W_kj, b_kj, W_ji, b_ji,
           W_bil, W_bs1, b_bs1, W_bs2, b_bs2, W_lin, b_lin,
           W_as1a, b_as1a, W_as1b, b_as1b, W_as2a, b_as2a, W_as2b, b_as2b):
    e = x.shape[0]
    t_n = idx_kj.shape[0]
    bf = jnp.bfloat16

    x_kj = _preamble(x, rbf, W_kj.astype(bf), b_kj.reshape(1, -1),
                     W_rbf.astype(bf))

    wt = jnp.transpose(W_bil, (1, 2, 0)).astype(bf)
    wsbf = W_sbf.astype(bf)

    unit = 16000
    n_units = t_n // unit
    base = n_units // _NC
    bounds = [0]
    for c in range(_NC):
        u = base + (1 if c < n_units % _NC else 0)
        bounds.append(bounds[-1] + u * unit)

    aggs = []
    for c in range(_NC):
        lo, hi = bounds[c], bounds[c + 1]
        g_c = _sc_gather(x_kj, idx_kj[lo:hi])
        t_c = _bilinear(g_c, sbf[lo:hi], wsbf, wt)
        aggs.append(_sc_scatter_add([t_c], [idx_ji[lo:hi]], e))

    mats = (W_ji, W_bs1, W_bs2, W_lin, W_as1a, W_as1b, W_as2a, W_as2b)
    biases = (b_ji, b_bs1, b_bs2, b_lin, b_as1a, b_as1b, b_as2a, b_as2b)
    return _final(aggs, x, tuple(m.astype(bf) for m in mats),
                  tuple(b.reshape(1, -1) for b in biases))


# speedup vs baseline: 1.0955x; 1.0007x over previous
import functools

import jax
import jax.numpy as jnp
from jax.experimental import pallas as pl
from jax.experimental.pallas import tpu as pltpu
from jax.experimental.pallas import tpu_sc as plsc

E_BLK = 2000
T_BLK = 2000
_NC = 2

_SC_MESH = plsc.VectorSubcoreMesh(core_axis_name="c", subcore_axis_name="s")


def _sc_gather(table, idx):
    t_n = idx.shape[0]
    h = table.shape[1]
    w = 128
    idx2 = idx.reshape(1, t_n)

    @functools.partial(
        pl.kernel,
        out_type=jax.ShapeDtypeStruct((t_n, h), jnp.float32),
        mesh=_SC_MESH,
    )
    def k(x_hbm, i_hbm, o_hbm):
        def body(i_vmem, o_vmem):
            pltpu.sync_copy(x_hbm.at[i_vmem.at[0]], o_vmem)

        pltpu.emit_pipeline(
            body,
            grid=(t_n // w,),
            in_specs=[pl.BlockSpec((1, w), index_map=lambda i: (0, i))],
            out_specs=[pl.BlockSpec((w, h), index_map=lambda i: (i, 0))],
            core_axis_name=("c", "s"),
            dimension_semantics=(pltpu.PARALLEL,),
        )(i_hbm, o_hbm)

    return k(table, idx2)


def _silu(v):
    return v / (1.0 + jnp.exp(-v))


def _mm(a, w):
    return jax.lax.dot_general(
        a.astype(jnp.bfloat16), w, (((1,), (0,)), ((), ())),
        preferred_element_type=jnp.float32)


_GW = 16
_IW = 128
_NIW = 5
_MW = _IW * _NIW


def _sc_scatter_add(ts, idxs, e):
    nchunk = len(ts)
    h = ts[0].shape[1]
    n_groups_per_core = h // (2 * _GW)
    nwins = [t.shape[0] // _MW for t in ts]
    stripe = e // 16
    zrows = stripe // ((stripe + _MW - 1) // _MW)
    assert stripe % zrows == 0 and zrows <= _MW
    nks = []
    for nwin in nwins:
        nk = (nwin + 15) // 16
        nks.append(nk + (nk % 2))
    idx2s = [idx.reshape(nw, _NIW, _IW) for idx, nw in zip(idxs, nwins)]

    @functools.partial(
        pl.kernel,
        out_type=jax.ShapeDtypeStruct((e, h), jnp.float32),
        mesh=_SC_MESH,
        compiler_params=pltpu.CompilerParams(use_tc_tiling_on_sc=False),
        scratch_types=[
            pltpu.VMEM_SHARED((e, _GW), jnp.float32),
            pltpu.VMEM((_NIW, _IW), jnp.int32),
            pltpu.VMEM((_NIW, _IW), jnp.int32),
            pltpu.VMEM((_MW, _GW), jnp.float32),
            pltpu.VMEM((_MW, _GW), jnp.float32),
            pltpu.SemaphoreType.DMA,
            pltpu.SemaphoreType.DMA,
            pltpu.SemaphoreType.DMA,
            pltpu.SemaphoreType.DMA,
        ],
    )
    def k(*refs):
        t_hbms = refs[:nchunk]
        i_hbms = refs[nchunk:2 * nchunk]
        (agg_hbm, acc, idx_v0, idx_v1, upd_v0, upd_v1,
         sem_i0, sem_i1, sem_u0, sem_u1) = refs[2 * nchunk:]
        c = jax.lax.axis_index("c")
        s = jax.lax.axis_index("s")
        idx_v = (idx_v0, idx_v1)
        upd_v = (upd_v0, upd_v1)
        sem_i = (sem_i0, sem_i1)
        sem_u = (sem_u0, sem_u1)
        base = s * stripe

        for g in range(n_groups_per_core):
            col = (c * n_groups_per_core + g) * _GW

            @pl.loop(0, zrows)
            def _(r):
                upd_v0[r, :] = jnp.zeros((_GW,), jnp.float32)

            for z in range(stripe // zrows):
                pltpu.async_copy(upd_v0.at[pl.ds(0, zrows), :],
                                 acc.at[pl.ds(base + z * zrows, zrows), :],
                                 sem_u0)
            for z in range(stripe // zrows):
                pltpu.make_async_copy(
                    upd_v0.at[pl.ds(0, zrows), :],
                    acc.at[pl.ds(base + z * zrows, zrows), :], sem_u0).wait()
            plsc.subcore_barrier()

            for t_hbm, i_hbm, nwin, nk in zip(t_hbms, i_hbms, nwins, nks):
                def fetch(b, w, t_hbm=t_hbm, i_hbm=i_hbm):
                    pltpu.async_copy(i_hbm.at[w], idx_v[b], sem_i[b])
                    pltpu.async_copy(
                        t_hbm.at[pl.ds(w * _MW, _MW), pl.ds(col, _GW)],
                        upd_v[b], sem_u[b])

                fetch(0, s)

                @pl.loop(0, nk, step=2)
                def _(kk, t_hbm=t_hbm, i_hbm=i_hbm, nwin=nwin, fetch=fetch):
                    for b in range(2):
                        w = (kk + b) * 16 + s
                        nxt = (kk + b + 1) * 16 + s

                        @pl.when(w < nwin)
                        def _():
                            pltpu.make_async_copy(
                                i_hbm.at[w], idx_v[b], sem_i[b]).wait()
                            pltpu.make_async_copy(
                                t_hbm.at[pl.ds(w * _MW, _MW), pl.ds(col, _GW)],
                                upd_v[b], sem_u[b]).wait()

                            @pl.when(nxt < nwin)
                            def _():
                                fetch(1 - b, nxt)

                            for j in range(_NIW):
                                pltpu.sync_copy(
                                    upd_v[b].at[pl.ds(j * _IW, _IW), :],
                                    acc.at[idx_v[b].at[j]], add=True)

            plsc.subcore_barrier()
            pltpu.sync_copy(acc.at[pl.ds(base, stripe), :],
                            agg_hbm.at[pl.ds(base, stripe), pl.ds(col, _GW)])
            if g == n_groups_per_core - 1:
                plsc.subcore_barrier()

    return k(*ts, *idx2s)


def _pre_body(x_ref, rbf_ref, wkj_ref, bkj_ref, wrbf_ref, xkj_ref):
    h = _silu(_mm(x_ref[...], wkj_ref[...]) + bkj_ref[...])
    rp = _mm(rbf_ref[...], wrbf_ref[...])
    xkj_ref[...] = h * rp


def _preamble(x, rbf, wkj, bkj, wrbf):
    e, h = x.shape
    nr = rbf.shape[1]
    nb = e // E_BLK
    return pl.pallas_call(
        _pre_body,
        grid=(nb,),
        in_specs=[
            pl.BlockSpec((E_BLK, h), lambda i: (i, 0)),
            pl.BlockSpec((E_BLK, nr), lambda i: (i, 0)),
            pl.BlockSpec((h, h), lambda i: (0, 0)),
            pl.BlockSpec((1, h), lambda i: (0, 0)),
            pl.BlockSpec((nr, h), lambda i: (0, 0)),
        ],
        out_specs=pl.BlockSpec((E_BLK, h), lambda i: (i, 0)),
        out_shape=jax.ShapeDtypeStruct((e, h), jnp.float32),
    )(x, rbf, wkj, bkj, wrbf)


def _bil_body(g_ref, sbf_ref, wsbf_ref, wt_ref, t_ref):
    sp = _mm(sbf_ref[...], wsbf_ref[...])
    g = g_ref[...].astype(jnp.bfloat16)
    nb = wt_ref.shape[0]
    acc = None
    for j in range(nb):
        m = jax.lax.dot_general(g, wt_ref[j], (((1,), (0,)), ((), ())),
                                preferred_element_type=jnp.float32)
        term = sp[:, j:j + 1] * m
        acc = term if acc is None else acc + term
    t_ref[...] = acc


def _bilinear(g, sbf, wsbf, wt):
    t_n, h = g.shape
    nsr = sbf.shape[1]
    nb = wt.shape[0]
    blocks = t_n // T_BLK
    return pl.pallas_call(
        _bil_body,
        grid=(blocks,),
        in_specs=[
            pl.BlockSpec((T_BLK, h), lambda i: (i, 0)),
            pl.BlockSpec((T_BLK, nsr), lambda i: (i, 0)),
            pl.BlockSpec((nsr, nb), lambda i: (0, 0)),
            pl.BlockSpec((nb, h, h), lambda i: (0, 0, 0)),
        ],
        out_specs=pl.BlockSpec((T_BLK, h), lambda i: (i, 0)),
        out_shape=jax.ShapeDtypeStruct((t_n, h), jnp.float32),
    )(g, sbf, wsbf, wt)


def _fin_body(n_agg, *refs):
    agg_refs = refs[:n_agg]
    (x_ref, wji_ref, bji_ref, w1_ref, b1_ref, w2_ref, b2_ref,
     wl_ref, bl_ref, wa_ref, ba_ref, wb_ref, bb_ref, wc_ref, bc_ref,
     wd_ref, bd_ref, h_ref) = refs[n_agg:]
    x = x_ref[...]
    x_ji = _silu(_mm(x, wji_ref[...]) + bji_ref[...])
    agg = agg_refs[0][...]
    for r in agg_refs[1:]:
        agg = agg + r[...]
    h = x_ji + agg
    h = h + _silu(_mm(_silu(_mm(h, w1_ref[...]) + b1_ref[...]), w2_ref[...])
                  + b2_ref[...])
    h = _silu(_mm(h, wl_ref[...]) + bl_ref[...]) + x
    h = h + _silu(_mm(_silu(_mm(h, wa_ref[...]) + ba_ref[...]), wb_ref[...])
                  + bb_ref[...])
    h = h + _silu(_mm(_silu(_mm(h, wc_ref[...]) + bc_ref[...]), wd_ref[...])
                  + bd_ref[...])
    h_ref[...] = h


def _final(aggs, x, mats, biases):
    e, h = x.shape
    nb = e // E_BLK
    mat_spec = pl.BlockSpec((h, h), lambda i: (0, 0))
    bias_spec = pl.BlockSpec((1, h), lambda i: (0, 0))
    edge_spec = pl.BlockSpec((E_BLK, h), lambda i: (i, 0))
    in_specs = [edge_spec] * (len(aggs) + 1)
    args = list(aggs) + [x]
    for m, b in zip(mats, biases):
        in_specs += [mat_spec, bias_spec]
        args += [m, b]
    return pl.pallas_call(
        functools.partial(_fin_body, len(aggs)),
        grid=(nb,),
        in_specs=in_specs,
        out_specs=pl.BlockSpec((E_BLK, h), lambda i: (i, 0)),
        out_shape=jax.ShapeDtypeStruct((e, h), jnp.float32),
    )(*args)


def kernel(x, rbf, sbf, idx_kj, idx_ji, W_rbf, W_sbf, W_kj, b_kj, W_ji, b_ji,
           W_bil, W_bs1, b_bs1, W_bs2, b_bs2, W_lin, b_lin,
           W_as1a, b_as1a, W_as1b, b_as1b, W_as2a, b_as2a, W_as2b, b_as2b):
    e = x.shape[0]
    t_n = idx_kj.shape[0]
    bf = jnp.bfloat16

    x_kj = _preamble(x, rbf, W_kj.astype(bf), b_kj.reshape(1, -1),
                     W_rbf.astype(bf))

    wt = jnp.transpose(W_bil, (1, 2, 0)).astype(bf)
    wsbf = W_sbf.astype(bf)

    unit = 16000
    n_units = t_n // unit
    base = n_units // _NC
    bounds = [0]
    for c in range(_NC):
        u = base + (1 if c >= _NC - n_units % _NC else 0)
        bounds.append(bounds[-1] + u * unit)

    aggs = []
    for c in range(_NC):
        lo, hi = bounds[c], bounds[c + 1]
        g_c = _sc_gather(x_kj, idx_kj[lo:hi])
        t_c = _bilinear(g_c, sbf[lo:hi], wsbf, wt)
        aggs.append(_sc_scatter_add([t_c], [idx_ji[lo:hi]], e))

    mats = (W_ji, W_bs1, W_bs2, W_lin, W_as1a, W_as1b, W_as2a, W_as2b)
    biases = (b_ji, b_bs1, b_bs2, b_lin, b_as1a, b_as1b, b_as2a, b_as2b)
    return _final(aggs, x, tuple(m.astype(bf) for m in mats),
                  tuple(b.reshape(1, -1) for b in biases))
